# baseline (speedup 1.0000x reference)
import functools

import jax
import jax.numpy as jnp
from jax import lax
from jax.experimental import pallas as pl
from jax.experimental.pallas import tpu as pltpu
from jax.experimental.pallas import tpu_sc as plsc

H = 768
EPS = 1e-12

NC = 2
NS = 16
NW = NC * NS

TC_BLK = 2048

SC_CHUNK = 32
SC_NBUF = 4


def _sc_gather(word_table, input_ids):
    nb, s = input_ids.shape
    n_tok = nb * s
    b_per_w = n_tok // NW
    w_per_row = s // b_per_w
    nchunk = b_per_w // SC_CHUNK
    mesh = plsc.VectorSubcoreMesh(core_axis_name="c", subcore_axis_name="s")

    @functools.partial(
        pl.kernel,
        mesh=mesh,
        out_type=jax.ShapeDtypeStruct((nb, s, H), jnp.float32),
        scratch_types=[
            pltpu.VMEM((b_per_w,), jnp.int32),
            pltpu.VMEM((SC_NBUF, SC_CHUNK, H), jnp.float32),
            pltpu.SemaphoreType.DMA,
            pltpu.SemaphoreType.DMA,
        ],
    )
    def gather_kernel(table_hbm, idx_hbm, out_hbm, idx_v, rows_v, sem_in, sem_out):
        wid = lax.axis_index("s") * NC + lax.axis_index("c")
        row = wid // w_per_row
        col = (wid % w_per_row) * b_per_w
        pltpu.sync_copy(idx_hbm.at[row, pl.ds(col, b_per_w)], idx_v)

        def g(c):
            return pltpu.make_async_copy(
                table_hbm.at[idx_v.at[pl.ds(c * SC_CHUNK, SC_CHUNK)]],
                rows_v.at[c % SC_NBUF],
                sem_in,
            )

        def p(c):
            return pltpu.make_async_copy(
                rows_v.at[c % SC_NBUF],
                out_hbm.at[row, pl.ds(col + c * SC_CHUNK, SC_CHUNK)],
                sem_out,
            )

        g(0).start()
        g(1).start()
        for c in range(nchunk):
            g(c).wait()
            nxt = c + 2
            if nxt < nchunk:
                if nxt - SC_NBUF >= 0:
                    p(nxt - SC_NBUF).wait()
                g(nxt).start()
            p(c).start()
        for c in range(max(0, nchunk - SC_NBUF), nchunk):
            p(c).wait()

    return gather_kernel(word_table, input_ids)


def _tc_body(words_ref, pos_ref, ttf_ref, type_ref, gamma_ref, beta_ref, out_ref):
    j = pl.program_id(1)
    t0 = type_ref[0:1, :]
    t1 = type_ref[1:2, :]
    nb = ttf_ref.shape[0]
    onehot = (lax.broadcasted_iota(jnp.int32, (nb, 1), 0) == j).astype(jnp.float32)
    tt = lax.dot_general(ttf_ref[...], onehot, (((0,), (0,)), ((), ())))
    x = words_ref[0] + pos_ref[...] + t0 + tt * (t1 - t0)
    u = jnp.mean(x, axis=1, keepdims=True)
    xc = x - u
    s = jnp.mean(xc * xc, axis=1, keepdims=True)
    y = xc * lax.rsqrt(s + EPS)
    out_ref[0] = (gamma_ref[...] * y + beta_ref[...]).reshape(y.shape)


def _tc_dense(words, pos_table, ttf, type_table, gamma, beta):
    nb, s, _ = words.shape
    pos_blocks = s // TC_BLK
    return pl.pallas_call(
        _tc_body,
        grid=(pos_blocks, nb),
        in_specs=[
            pl.BlockSpec((1, TC_BLK, H), lambda p, j: (j, p, 0)),
            pl.BlockSpec((TC_BLK, H), lambda p, j: (p, 0)),
            pl.BlockSpec((nb, TC_BLK), lambda p, j: (0, p)),
            pl.BlockSpec((2, H), lambda p, j: (0, 0)),
            pl.BlockSpec((H,), lambda p, j: (0,)),
            pl.BlockSpec((H,), lambda p, j: (0,)),
        ],
        out_specs=pl.BlockSpec((1, TC_BLK, H), lambda p, j: (j, p, 0)),
        out_shape=jax.ShapeDtypeStruct((nb, s, H), jnp.float32),
        compiler_params=pltpu.CompilerParams(
            dimension_semantics=("parallel", "parallel"),
        ),
    )(words, pos_table, ttf, type_table, gamma, beta)


def kernel(input_ids, token_type_ids, word_table, pos_table, type_table, gamma, beta):
    b, s = input_ids.shape
    ids = input_ids.astype(jnp.int32)
    ttf = token_type_ids.astype(jnp.float32)

    hb = b // 2
    w0 = _sc_gather(word_table, ids[:hb])
    w1 = _sc_gather(word_table, ids[hb:])
    o0 = _tc_dense(w0, pos_table, ttf[:hb], type_table, gamma, beta)
    o1 = _tc_dense(w1, pos_table, ttf[hb:], type_table, gamma, beta)

    out = jnp.concatenate([o0, o1], axis=0)
    words = jnp.concatenate([w0, w1], axis=0)
    return (out, words)


# speedup vs baseline: 1.5443x; 1.5443x over previous
import functools

import jax
import jax.numpy as jnp
from jax import lax
from jax.experimental import pallas as pl
from jax.experimental.pallas import tpu as pltpu
from jax.experimental.pallas import tpu_sc as plsc

H = 768
EPS = 1e-12

NC = 2
NS = 16
NW = NC * NS

TC_BLK = 2048

SC_CHUNK = 32
SC_NBUF = 4


def _sc_gather(word_table, input_ids):
    nb, s = input_ids.shape
    n_tok = nb * s
    b_per_w = n_tok // NW
    w_per_row = s // b_per_w
    nchunk = b_per_w // SC_CHUNK
    mesh = plsc.VectorSubcoreMesh(core_axis_name="c", subcore_axis_name="s")

    @functools.partial(
        pl.kernel,
        mesh=mesh,
        out_type=jax.ShapeDtypeStruct((nb, s, H), jnp.float32),
        scratch_types=[
            pltpu.VMEM((b_per_w,), jnp.int32),
            pltpu.VMEM((SC_NBUF, SC_CHUNK, H), jnp.float32),
            pltpu.SemaphoreType.DMA,
            pltpu.SemaphoreType.DMA,
        ],
    )
    def gather_kernel(table_hbm, idx_hbm, out_hbm, idx_v, rows_v, sem_in, sem_out):
        wid = lax.axis_index("s") * NC + lax.axis_index("c")
        row = wid // w_per_row
        col = (wid % w_per_row) * b_per_w
        pltpu.sync_copy(idx_hbm.at[row, pl.ds(col, b_per_w)], idx_v)

        def g(c):
            return pltpu.make_async_copy(
                table_hbm.at[idx_v.at[pl.ds(c * SC_CHUNK, SC_CHUNK)]],
                rows_v.at[c % SC_NBUF],
                sem_in,
            )

        def p(c):
            return pltpu.make_async_copy(
                rows_v.at[c % SC_NBUF],
                out_hbm.at[row, pl.ds(col + c * SC_CHUNK, SC_CHUNK)],
                sem_out,
            )

        g(0).start()
        g(1).start()
        g(2).start()
        for c in range(nchunk):
            g(c).wait()
            nxt = c + 3
            if nxt < nchunk:
                if nxt - SC_NBUF >= 0:
                    p(nxt - SC_NBUF).wait()
                g(nxt).start()
            p(c).start()
        for c in range(max(0, nchunk - SC_NBUF), nchunk):
            p(c).wait()

    return gather_kernel(word_table, input_ids)


def _tc_body(words_ref, pos_ref, ttf_ref, type_ref, gamma_ref, beta_ref, out_ref):
    j = pl.program_id(1)
    t0 = type_ref[0:1, :]
    t1 = type_ref[1:2, :]
    nb = ttf_ref.shape[0]
    onehot = (lax.broadcasted_iota(jnp.int32, (nb, 1), 0) == j).astype(jnp.float32)
    tt = lax.dot_general(ttf_ref[...], onehot, (((0,), (0,)), ((), ())))
    x = words_ref[0] + pos_ref[...] + t0 + tt * (t1 - t0)
    u = jnp.mean(x, axis=1, keepdims=True)
    xc = x - u
    s = jnp.mean(xc * xc, axis=1, keepdims=True)
    y = xc * lax.rsqrt(s + EPS)
    out_ref[0] = (gamma_ref[...] * y + beta_ref[...]).reshape(y.shape)


def kernel(input_ids, token_type_ids, word_table, pos_table, type_table, gamma, beta):
    b, s = input_ids.shape

    words = _sc_gather(word_table, input_ids.astype(jnp.int32))

    ttf = token_type_ids.astype(jnp.float32)
    pos_blocks = s // TC_BLK

    out = pl.pallas_call(
        _tc_body,
        grid=(pos_blocks, b),
        in_specs=[
            pl.BlockSpec((1, TC_BLK, H), lambda p, j: (j, p, 0)),
            pl.BlockSpec((TC_BLK, H), lambda p, j: (p, 0)),
            pl.BlockSpec((b, TC_BLK), lambda p, j: (0, p)),
            pl.BlockSpec((2, H), lambda p, j: (0, 0)),
            pl.BlockSpec((H,), lambda p, j: (0,)),
            pl.BlockSpec((H,), lambda p, j: (0,)),
        ],
        out_specs=pl.BlockSpec((1, TC_BLK, H), lambda p, j: (j, p, 0)),
        out_shape=jax.ShapeDtypeStruct((b, s, H), jnp.float32),
        compiler_params=pltpu.CompilerParams(
            dimension_semantics=("parallel", "parallel"),
        ),
    )(
        words,
        pos_table,
        ttf,
        type_table,
        gamma,
        beta,
    )

    return (out, words)


# speedup vs baseline: 1.5483x; 1.0026x over previous
import functools

import jax
import jax.numpy as jnp
from jax import lax
from jax.experimental import pallas as pl
from jax.experimental.pallas import tpu as pltpu
from jax.experimental.pallas import tpu_sc as plsc

H = 768
EPS = 1e-12

NC = 2
NS = 16
NW = NC * NS

TC_BLK = 2048

SC_CHUNK = 32
SC_NBUF = 4


def _sc_gather(word_table, input_ids):
    nb, s = input_ids.shape
    n_tok = nb * s
    b_per_w = n_tok // NW
    w_per_row = s // b_per_w
    nchunk = b_per_w // SC_CHUNK
    mesh = plsc.VectorSubcoreMesh(core_axis_name="c", subcore_axis_name="s")

    @functools.partial(
        pl.kernel,
        mesh=mesh,
        out_type=jax.ShapeDtypeStruct((nb, s, H), jnp.float32),
        scratch_types=[
            pltpu.VMEM((b_per_w,), jnp.int32),
            pltpu.VMEM((SC_NBUF, SC_CHUNK, H), jnp.float32),
            pltpu.SemaphoreType.DMA,
            pltpu.SemaphoreType.DMA,
        ],
    )
    def gather_kernel(table_hbm, idx_hbm, out_hbm, idx_v, rows_v, sem_in, sem_out):
        wid = lax.axis_index("s") * NC + lax.axis_index("c")
        row = wid // w_per_row
        col = (wid % w_per_row) * b_per_w
        pltpu.sync_copy(idx_hbm.at[row, pl.ds(col, b_per_w)], idx_v)

        def g(c):
            return pltpu.make_async_copy(
                table_hbm.at[idx_v.at[pl.ds(c * SC_CHUNK, SC_CHUNK)]],
                rows_v.at[c % SC_NBUF],
                sem_in,
            )

        def p(c):
            return pltpu.make_async_copy(
                rows_v.at[c % SC_NBUF],
                out_hbm.at[row, pl.ds(col + c * SC_CHUNK, SC_CHUNK)],
                sem_out,
            )

        g(0).start()
        g(1).start()
        for c in range(nchunk):
            g(c).wait()
            nxt = c + 2
            if nxt < nchunk:
                if nxt - SC_NBUF >= 0:
                    p(nxt - SC_NBUF).wait()
                g(nxt).start()
            p(c).start()
        for c in range(max(0, nchunk - SC_NBUF), nchunk):
            p(c).wait()

    return gather_kernel(word_table, input_ids)


def _tc_body(words_ref, pos_ref, ttf_ref, type_ref, gamma_ref, beta_ref, out_ref):
    j = pl.program_id(1)
    t0 = type_ref[0:1, :]
    t1 = type_ref[1:2, :]
    nb = ttf_ref.shape[0]
    onehot = (lax.broadcasted_iota(jnp.int32, (nb, 1), 0) == j).astype(jnp.float32)
    tt = lax.dot_general(ttf_ref[...], onehot, (((0,), (0,)), ((), ())))
    x = words_ref[0] + pos_ref[...] + t0 + tt * (t1 - t0)
    u = jnp.mean(x, axis=1, keepdims=True)
    xc = x - u
    s = jnp.mean(xc * xc, axis=1, keepdims=True)
    y = xc * lax.rsqrt(s + EPS)
    out_ref[0] = (gamma_ref[...] * y + beta_ref[...]).reshape(y.shape)


def kernel(input_ids, token_type_ids, word_table, pos_table, type_table, gamma, beta):
    b, s = input_ids.shape

    words = _sc_gather(word_table, input_ids.astype(jnp.int32))

    ttf = token_type_ids.astype(jnp.float32)
    pos_blocks = s // TC_BLK

    out = pl.pallas_call(
        _tc_body,
        grid=(pos_blocks, b),
        in_specs=[
            pl.BlockSpec((1, TC_BLK, H), lambda p, j: (j, p, 0)),
            pl.BlockSpec((TC_BLK, H), lambda p, j: (p, 0)),
            pl.BlockSpec((b, TC_BLK), lambda p, j: (0, p)),
            pl.BlockSpec((2, H), lambda p, j: (0, 0)),
            pl.BlockSpec((H,), lambda p, j: (0,)),
            pl.BlockSpec((H,), lambda p, j: (0,)),
        ],
        out_specs=pl.BlockSpec((1, TC_BLK, H), lambda p, j: (j, p, 0)),
        out_shape=jax.ShapeDtypeStruct((b, s, H), jnp.float32),
        compiler_params=pltpu.CompilerParams(
            dimension_semantics=("parallel", "parallel"),
        ),
    )(
        words,
        pos_table,
        ttf,
        type_table,
        gamma,
        beta,
    )

    return (out, words)
